# baseline (speedup 1.0000x reference)
import functools

import jax
import jax.numpy as jnp
from jax import lax
from jax.experimental import pallas as pl
from jax.experimental.pallas import tpu as pltpu
from jax.experimental.pallas import tpu_sc as plsc

NC = 2
NS = 16
NW = NC * NS
CH = 128
EPS = 1e-8


def _matmul_call(xp, w):
  nr, d = xp.shape
  o = w.shape[0]
  br = 1024

  def body(x_ref, w_ref, o_ref):
    o_ref[...] = lax.dot_general(
        x_ref[...], w_ref[...], (((1,), (1,)), ((), ())),
        preferred_element_type=jnp.float32)

  return pl.pallas_call(
      body,
      grid=(nr // br,),
      in_specs=[
          pl.BlockSpec((br, d), lambda i: (i, 0)),
          pl.BlockSpec((o, d), lambda i: (0, 0)),
      ],
      out_specs=pl.BlockSpec((br, o), lambda i: (i, 0)),
      out_shape=jax.ShapeDtypeStruct((nr, o), jnp.float32),
  )(xp, w)


@functools.lru_cache(maxsize=None)
def _sc_gather_scatter(nr, d, kt, kb, k0):
  rpt = nr // NS
  nz = rpt // CH
  mesh = plsc.VectorSubcoreMesh(
      core_axis_name="c", subcore_axis_name="s", num_cores=NC,
      num_subcores=NS)

  @functools.partial(
      pl.kernel,
      out_type=jax.ShapeDtypeStruct((NC, nr, d), jnp.float32),
      mesh=mesh,
      scratch_types=[
          pltpu.VMEM((kb, CH), jnp.int32),
          pltpu.VMEM((kb, CH), jnp.int32),
          pltpu.VMEM((CH, d), jnp.float32),
          pltpu.VMEM((CH, d), jnp.float32),
          pltpu.VMEM_SHARED((nr, d), jnp.float32),
          pltpu.SemaphoreType.DMA,
          pltpu.SemaphoreType.DMA,
          pltpu.SemaphoreType.DMA,
          pltpu.SemaphoreType.DMA,
      ],
  )
  def sck(table, gidx, sidx, psum, gv, sv, bufa, bufb, acc,
          semga, semgb, semsa, semsb):
    c = lax.axis_index("c")
    s = lax.axis_index("s")
    koff = c * k0

    def fill_row(i, _):
      def fill16(j, _):
        bufa[i, pl.ds(j * 16, 16)] = jnp.zeros((16,), jnp.float32)
        return 0
      lax.fori_loop(0, d // 16, fill16, 0)
      return 0

    lax.fori_loop(0, CH, fill_row, 0)

    base = s * rpt
    for z in range(nz):
      pltpu.sync_copy(bufa, acc.at[pl.ds(base + z * CH, CH)])
    plsc.subcore_barrier()

    def gather(t, buf, sem):
      pltpu.async_copy(table.at[gv.at[t]], buf, sem)

    def drain(buf, sem):
      pltpu.make_async_copy(table.at[pl.ds(0, CH)], buf, sem).wait()

    def scatter(t, buf, sem):
      pltpu.async_copy(buf, acc.at[sv.at[t]], sem, add=True)

    def batch(b, _):
      pltpu.sync_copy(gidx.at[s, pl.ds(koff + b * kb, kb)], gv)
      pltpu.sync_copy(sidx.at[s, pl.ds(koff + b * kb, kb)], sv)
      gather(0, bufa, semga)

      def pair(i, _):
        t0 = 2 * i

        @pl.when(i > 0)
        def _():
          drain(bufb, semsb)

        gather(t0 + 1, bufb, semgb)
        drain(bufa, semga)
        scatter(t0, bufa, semsa)

        @pl.when(t0 + 2 < kb)
        def _():
          drain(bufa, semsa)
          gather(t0 + 2, bufa, semga)

        drain(bufb, semgb)
        scatter(t0 + 1, bufb, semsb)
        return 0

      lax.fori_loop(0, kb // 2, pair, 0)
      drain(bufa, semsa)
      drain(bufb, semsb)
      return 0

    @pl.when(c == 0)
    def _():
      lax.fori_loop(0, k0 // kb, batch, 0)

    @pl.when(c == 1)
    def _():
      lax.fori_loop(0, (kt - k0) // kb, batch, 0)

    plsc.subcore_barrier()

    for z in range(nz):
      pltpu.sync_copy(acc.at[pl.ds(base + z * CH, CH)], bufa)
      pltpu.sync_copy(bufa, psum.at[c, pl.ds(base + z * CH, CH)])

  return sck


@functools.lru_cache(maxsize=None)
def _sc_counts(nr, kt, kb, k0):
  rpt = nr // NS
  nz = rpt // CH
  mesh = plsc.VectorSubcoreMesh(
      core_axis_name="c", subcore_axis_name="s", num_cores=NC,
      num_subcores=NS)

  @functools.partial(
      pl.kernel,
      out_type=(
          jax.ShapeDtypeStruct((NC, nr, 16), jnp.float32),
          jax.ShapeDtypeStruct((NC, nr, 16), jnp.float32),
      ),
      mesh=mesh,
      compiler_params=pltpu.CompilerParams(use_tc_tiling_on_sc=False),
      scratch_types=[
          pltpu.VMEM((kb, CH), jnp.int32),
          pltpu.VMEM((kb, CH), jnp.int32),
          pltpu.VMEM((CH, 16), jnp.float32),
          pltpu.VMEM((CH, 16), jnp.float32),
          pltpu.VMEM_SHARED((nr, 16), jnp.float32),
          pltpu.VMEM_SHARED((nr, 16), jnp.float32),
      ],
  )
  def sck(didx, sidx, cntd, cnts, dv, sv, ones, zrow, accd, accs):
    c = lax.axis_index("c")
    s = lax.axis_index("s")
    koff = c * k0

    def fill_row(i, _):
      ones[i] = jnp.full((16,), 1.0, jnp.float32)
      zrow[i] = jnp.zeros((16,), jnp.float32)
      return 0

    lax.fori_loop(0, CH, fill_row, 0)

    base = s * rpt
    for z in range(nz):
      pltpu.sync_copy(zrow, accd.at[pl.ds(base + z * CH, CH)])
      pltpu.sync_copy(zrow, accs.at[pl.ds(base + z * CH, CH)])
    plsc.subcore_barrier()

    def step(j, _):
      pltpu.sync_copy(ones, accd.at[dv.at[j]], add=True)
      pltpu.sync_copy(ones, accs.at[sv.at[j]], add=True)
      return 0

    def batch(b, _):
      pltpu.sync_copy(didx.at[s, pl.ds(koff + b * kb, kb)], dv)
      pltpu.sync_copy(sidx.at[s, pl.ds(koff + b * kb, kb)], sv)
      lax.fori_loop(0, kb, step, 0)
      return 0

    @pl.when(c == 0)
    def _():
      lax.fori_loop(0, k0 // kb, batch, 0)

    @pl.when(c == 1)
    def _():
      lax.fori_loop(0, (kt - k0) // kb, batch, 0)

    plsc.subcore_barrier()

    for z in range(nz):
      pltpu.sync_copy(accd.at[pl.ds(base + z * CH, CH)], ones)
      pltpu.sync_copy(ones, cntd.at[c, pl.ds(base + z * CH, CH)])
      pltpu.sync_copy(accs.at[pl.ds(base + z * CH, CH)], zrow)
      pltpu.sync_copy(zrow, cnts.at[c, pl.ds(base + z * CH, CH)])

  return sck


def _combine_norm_call(p, pc):
  nr, d = p.shape[1], p.shape[2]
  br = 1024

  def body(p_ref, c_ref, o_ref):
    ssum = p_ref[0] + p_ref[1]
    cnt = c_ref[0, :, 0:1] + c_ref[1, :, 0:1]
    o_ref[...] = ssum / (cnt + EPS)

  return pl.pallas_call(
      body,
      grid=(nr // br,),
      in_specs=[
          pl.BlockSpec((NC, br, d), lambda i: (0, i, 0)),
          pl.BlockSpec((NC, br, 16), lambda i: (0, i, 0)),
      ],
      out_specs=pl.BlockSpec((br, d), lambda i: (i, 0)),
      out_shape=jax.ShapeDtypeStruct((nr, d), jnp.float32),
  )(p, pc)


def _final_call(q, qc, xn, bias2d):
  nr, d = q.shape[1], q.shape[2]
  br = 1024

  def body(q_ref, c_ref, x_ref, b_ref, o_ref):
    ssum = q_ref[0] + q_ref[1]
    deg = c_ref[0, :, 0:1] + c_ref[1, :, 0:1]
    o_ref[...] = ssum / (deg + EPS) + x_ref[...] + b_ref[...]

  return pl.pallas_call(
      body,
      grid=(nr // br,),
      in_specs=[
          pl.BlockSpec((NC, br, d), lambda i: (0, i, 0)),
          pl.BlockSpec((NC, br, 16), lambda i: (0, i, 0)),
          pl.BlockSpec((br, d), lambda i: (i, 0)),
          pl.BlockSpec((1, d), lambda i: (0, 0)),
      ],
      out_specs=pl.BlockSpec((br, d), lambda i: (i, 0)),
      out_shape=jax.ShapeDtypeStruct((nr, d), jnp.float32),
  )(q, qc, xn, bias2d)


def kernel(x, hyperedge_index, W_node, W_edge, bias):
  n, d = x.shape
  src = hyperedge_index[0].astype(jnp.int32)
  dst = hyperedge_index[1].astype(jnp.int32)
  ni = src.shape[0]

  nr = -(-(n + 1) // (NS * CH)) * (NS * CH)
  kb = 16
  kt = -(-ni // (NS * CH * kb)) * kb
  k0 = (kt * 9 // 10) // kb * kb
  npad = NS * kt * CH - ni
  src_p = jnp.concatenate(
      [src, jnp.full((npad,), n, jnp.int32)]).reshape(NS, kt, CH)
  dst_p = jnp.concatenate(
      [dst, jnp.full((npad,), n, jnp.int32)]).reshape(NS, kt, CH)

  xp = jnp.zeros((nr, d), jnp.float32).at[:n].set(x)
  xn = _matmul_call(xp, W_node)

  sck = _sc_gather_scatter(nr, d, kt, kb, k0)
  p = sck(xn, src_p, dst_p)
  cnt, deg = _sc_counts(nr, kt, kb, kt // 2)(dst_p, src_p)
  attr2 = _combine_norm_call(p, cnt)
  q = sck(attr2, dst_p, src_p)
  out = _final_call(q, deg, xn, bias.reshape(1, d))
  return out[:n]


# speedup vs baseline: 1.1119x; 1.1119x over previous
import functools

import jax
import jax.numpy as jnp
from jax import lax
from jax.experimental import pallas as pl
from jax.experimental.pallas import tpu as pltpu
from jax.experimental.pallas import tpu_sc as plsc

NC = 2
NS = 16
NW = NC * NS
CH = 128
EPS = 1e-8


def _matmul_call(xp, w):
  nr, d = xp.shape
  o = w.shape[0]
  br = 1024

  def body(x_ref, w_ref, o_ref):
    o_ref[...] = lax.dot_general(
        x_ref[...], w_ref[...], (((1,), (1,)), ((), ())),
        preferred_element_type=jnp.float32)

  return pl.pallas_call(
      body,
      grid=(nr // br,),
      in_specs=[
          pl.BlockSpec((br, d), lambda i: (i, 0)),
          pl.BlockSpec((o, d), lambda i: (0, 0)),
      ],
      out_specs=pl.BlockSpec((br, o), lambda i: (i, 0)),
      out_shape=jax.ShapeDtypeStruct((nr, o), jnp.float32),
  )(xp, w)


@functools.lru_cache(maxsize=None)
def _sc_gather_scatter(nr, d, kt, kb, k0):
  rpt = nr // NS
  nz = rpt // CH
  mesh = plsc.VectorSubcoreMesh(
      core_axis_name="c", subcore_axis_name="s", num_cores=NC,
      num_subcores=NS)

  @functools.partial(
      pl.kernel,
      out_type=jax.ShapeDtypeStruct((NC, nr, d), jnp.float32),
      mesh=mesh,
      scratch_types=[
          pltpu.VMEM((kb, CH), jnp.int32),
          pltpu.VMEM((kb, CH), jnp.int32),
          pltpu.VMEM((CH, d), jnp.float32),
          pltpu.VMEM((CH, d), jnp.float32),
          pltpu.VMEM_SHARED((nr, d), jnp.float32),
          pltpu.SemaphoreType.DMA,
          pltpu.SemaphoreType.DMA,
          pltpu.SemaphoreType.DMA,
          pltpu.SemaphoreType.DMA,
      ],
  )
  def sck(table, gidx, sidx, psum, gv, sv, bufa, bufb, acc,
          semga, semgb, semsa, semsb):
    c = lax.axis_index("c")
    s = lax.axis_index("s")
    koff = c * k0

    def fill_row(i, _):
      def fill16(j, _):
        bufa[i, pl.ds(j * 16, 16)] = jnp.zeros((16,), jnp.float32)
        return 0
      lax.fori_loop(0, d // 16, fill16, 0)
      return 0

    lax.fori_loop(0, CH, fill_row, 0)

    base = s * rpt
    for z in range(nz):
      pltpu.sync_copy(bufa, acc.at[pl.ds(base + z * CH, CH)])
    plsc.subcore_barrier()

    def gather(t, buf, sem):
      pltpu.async_copy(table.at[gv.at[t]], buf, sem)

    def drain(buf, sem):
      pltpu.make_async_copy(table.at[pl.ds(0, CH)], buf, sem).wait()

    def scatter(t, buf, sem):
      pltpu.async_copy(buf, acc.at[sv.at[t]], sem, add=True)

    def batch(b, _):
      pltpu.sync_copy(gidx.at[s, pl.ds(koff + b * kb, kb)], gv)
      pltpu.sync_copy(sidx.at[s, pl.ds(koff + b * kb, kb)], sv)
      gather(0, bufa, semga)

      def pair(i, _):
        t0 = 2 * i

        @pl.when(i > 0)
        def _():
          drain(bufb, semsb)

        gather(t0 + 1, bufb, semgb)
        drain(bufa, semga)
        scatter(t0, bufa, semsa)

        @pl.when(t0 + 2 < kb)
        def _():
          drain(bufa, semsa)
          gather(t0 + 2, bufa, semga)

        drain(bufb, semgb)
        scatter(t0 + 1, bufb, semsb)
        return 0

      lax.fori_loop(0, kb // 2, pair, 0)
      drain(bufa, semsa)
      drain(bufb, semsb)
      return 0

    @pl.when(c == 0)
    def _():
      lax.fori_loop(0, k0 // kb, batch, 0)

    @pl.when(c == 1)
    def _():
      lax.fori_loop(0, (kt - k0) // kb, batch, 0)

    plsc.subcore_barrier()

    for z in range(nz):
      pltpu.sync_copy(acc.at[pl.ds(base + z * CH, CH)], bufa)
      pltpu.sync_copy(bufa, psum.at[c, pl.ds(base + z * CH, CH)])

  return sck


@functools.lru_cache(maxsize=None)
def _sc_counts(nr, kt, kb, k0):
  rpt = nr // NS
  nz = rpt // CH
  mesh = plsc.VectorSubcoreMesh(
      core_axis_name="c", subcore_axis_name="s", num_cores=NC,
      num_subcores=NS)

  @functools.partial(
      pl.kernel,
      out_type=(
          jax.ShapeDtypeStruct((NC, nr, 16), jnp.float32),
          jax.ShapeDtypeStruct((NC, nr, 16), jnp.float32),
      ),
      mesh=mesh,
      compiler_params=pltpu.CompilerParams(use_tc_tiling_on_sc=False),
      scratch_types=[
          pltpu.VMEM((kb, CH), jnp.int32),
          pltpu.VMEM((kb, CH), jnp.int32),
          pltpu.VMEM((CH, 16), jnp.float32),
          pltpu.VMEM((CH, 16), jnp.float32),
          pltpu.VMEM_SHARED((nr, 16), jnp.float32),
          pltpu.VMEM_SHARED((nr, 16), jnp.float32),
      ],
  )
  def sck(didx, sidx, cntd, cnts, dv, sv, ones, zrow, accd, accs):
    c = lax.axis_index("c")
    s = lax.axis_index("s")
    koff = c * k0

    def fill_row(i, _):
      ones[i] = jnp.full((16,), 1.0, jnp.float32)
      zrow[i] = jnp.zeros((16,), jnp.float32)
      return 0

    lax.fori_loop(0, CH, fill_row, 0)

    base = s * rpt
    for z in range(nz):
      pltpu.sync_copy(zrow, accd.at[pl.ds(base + z * CH, CH)])
      pltpu.sync_copy(zrow, accs.at[pl.ds(base + z * CH, CH)])
    plsc.subcore_barrier()

    def step(j, _):
      pltpu.sync_copy(ones, accd.at[dv.at[j]], add=True)
      pltpu.sync_copy(ones, accs.at[sv.at[j]], add=True)
      return 0

    def batch(b, _):
      pltpu.sync_copy(didx.at[s, pl.ds(koff + b * kb, kb)], dv)
      pltpu.sync_copy(sidx.at[s, pl.ds(koff + b * kb, kb)], sv)
      lax.fori_loop(0, kb, step, 0)
      return 0

    @pl.when(c == 0)
    def _():
      lax.fori_loop(0, k0 // kb, batch, 0)

    @pl.when(c == 1)
    def _():
      lax.fori_loop(0, (kt - k0) // kb, batch, 0)

    plsc.subcore_barrier()

    for z in range(nz):
      pltpu.sync_copy(accd.at[pl.ds(base + z * CH, CH)], ones)
      pltpu.sync_copy(ones, cntd.at[c, pl.ds(base + z * CH, CH)])
      pltpu.sync_copy(accs.at[pl.ds(base + z * CH, CH)], zrow)
      pltpu.sync_copy(zrow, cnts.at[c, pl.ds(base + z * CH, CH)])

  return sck


def _combine_norm_call(p, pc):
  nr, d = p.shape[1], p.shape[2]
  br = 1024

  def body(p_ref, c_ref, o_ref):
    ssum = p_ref[0] + p_ref[1]
    cnt = c_ref[0, :, 0:1] + c_ref[1, :, 0:1]
    o_ref[...] = ssum / (cnt + EPS)

  return pl.pallas_call(
      body,
      grid=(nr // br,),
      in_specs=[
          pl.BlockSpec((NC, br, d), lambda i: (0, i, 0)),
          pl.BlockSpec((NC, br, 16), lambda i: (0, i, 0)),
      ],
      out_specs=pl.BlockSpec((br, d), lambda i: (i, 0)),
      out_shape=jax.ShapeDtypeStruct((nr, d), jnp.float32),
  )(p, pc)


def _final_call(q, qc, xn, bias2d):
  nr, d = q.shape[1], q.shape[2]
  br = 1024

  def body(q_ref, c_ref, x_ref, b_ref, o_ref):
    ssum = q_ref[0] + q_ref[1]
    deg = c_ref[0, :, 0:1] + c_ref[1, :, 0:1]
    o_ref[...] = ssum / (deg + EPS) + x_ref[...] + b_ref[...]

  return pl.pallas_call(
      body,
      grid=(nr // br,),
      in_specs=[
          pl.BlockSpec((NC, br, d), lambda i: (0, i, 0)),
          pl.BlockSpec((NC, br, 16), lambda i: (0, i, 0)),
          pl.BlockSpec((br, d), lambda i: (i, 0)),
          pl.BlockSpec((1, d), lambda i: (0, 0)),
      ],
      out_specs=pl.BlockSpec((br, d), lambda i: (i, 0)),
      out_shape=jax.ShapeDtypeStruct((nr, d), jnp.float32),
  )(q, qc, xn, bias2d)


def kernel(x, hyperedge_index, W_node, W_edge, bias):
  n, d = x.shape
  src = hyperedge_index[0].astype(jnp.int32)
  dst = hyperedge_index[1].astype(jnp.int32)
  ni = src.shape[0]

  nr = -(-(n + 1) // (NS * CH)) * (NS * CH)
  kb = 16
  kt = -(-ni // (NS * CH * kb)) * kb
  k0 = (kt * 7 // 10) // kb * kb
  npad = NS * kt * CH - ni
  src_p = jnp.concatenate(
      [src, jnp.full((npad,), n, jnp.int32)]).reshape(NS, kt, CH)
  dst_p = jnp.concatenate(
      [dst, jnp.full((npad,), n, jnp.int32)]).reshape(NS, kt, CH)

  xp = jnp.zeros((nr, d), jnp.float32).at[:n].set(x)
  xn = _matmul_call(xp, W_node)

  sck = _sc_gather_scatter(nr, d, kt, kb, k0)
  p = sck(xn, src_p, dst_p)
  cnt, deg = _sc_counts(nr, kt, kb, kt // 2)(dst_p, src_p)
  attr2 = _combine_norm_call(p, cnt)
  q = sck(attr2, dst_p, src_p)
  out = _final_call(q, deg, xn, bias.reshape(1, d))
  return out[:n]


# speedup vs baseline: 1.1700x; 1.0522x over previous
import functools

import jax
import jax.numpy as jnp
from jax import lax
from jax.experimental import pallas as pl
from jax.experimental.pallas import tpu as pltpu
from jax.experimental.pallas import tpu_sc as plsc

NC = 2
NS = 16
NW = NC * NS
CH = 128
EPS = 1e-8


def _matmul_call(xp, w):
  nr, d = xp.shape
  o = w.shape[0]
  br = 1024

  def body(x_ref, w_ref, o_ref):
    o_ref[...] = lax.dot_general(
        x_ref[...], w_ref[...], (((1,), (1,)), ((), ())),
        preferred_element_type=jnp.float32)

  return pl.pallas_call(
      body,
      grid=(nr // br,),
      in_specs=[
          pl.BlockSpec((br, d), lambda i: (i, 0)),
          pl.BlockSpec((o, d), lambda i: (0, 0)),
      ],
      out_specs=pl.BlockSpec((br, o), lambda i: (i, 0)),
      out_shape=jax.ShapeDtypeStruct((nr, o), jnp.float32),
  )(xp, w)


@functools.lru_cache(maxsize=None)
def _sc_gather_scatter(nr, d, kt, kb, k0):
  rpt = nr // NS
  nz = rpt // CH
  mesh = plsc.VectorSubcoreMesh(
      core_axis_name="c", subcore_axis_name="s", num_cores=NC,
      num_subcores=NS)

  @functools.partial(
      pl.kernel,
      out_type=jax.ShapeDtypeStruct((NC, nr, d), jnp.float32),
      mesh=mesh,
      scratch_types=[
          pltpu.VMEM((kb, CH), jnp.int32),
          pltpu.VMEM((kb, CH), jnp.int32),
          pltpu.VMEM((CH, d), jnp.float32),
          pltpu.VMEM((CH, d), jnp.float32),
          pltpu.VMEM_SHARED((nr, d), jnp.float32),
          pltpu.SemaphoreType.DMA,
          pltpu.SemaphoreType.DMA,
          pltpu.SemaphoreType.DMA,
          pltpu.SemaphoreType.DMA,
      ],
  )
  def sck(table, gidx, sidx, psum, gv, sv, bufa, bufb, acc,
          semga, semgb, semsa, semsb):
    c = lax.axis_index("c")
    s = lax.axis_index("s")
    koff = c * k0

    def fill_row(i, _):
      def fill16(j, _):
        bufa[i, pl.ds(j * 16, 16)] = jnp.zeros((16,), jnp.float32)
        return 0
      lax.fori_loop(0, d // 16, fill16, 0)
      return 0

    lax.fori_loop(0, CH, fill_row, 0)

    base = s * rpt
    for z in range(nz):
      pltpu.sync_copy(bufa, acc.at[pl.ds(base + z * CH, CH)])
    plsc.subcore_barrier()

    def gather(t, buf, sem):
      pltpu.async_copy(table.at[gv.at[t]], buf, sem)

    def drain(buf, sem):
      pltpu.make_async_copy(table.at[pl.ds(0, CH)], buf, sem).wait()

    def scatter(t, buf, sem):
      pltpu.async_copy(buf, acc.at[sv.at[t]], sem, add=True)

    def batch(b, _):
      pltpu.sync_copy(gidx.at[s, pl.ds(koff + b * kb, kb)], gv)
      pltpu.sync_copy(sidx.at[s, pl.ds(koff + b * kb, kb)], sv)
      gather(0, bufa, semga)

      def pair(i, _):
        t0 = 2 * i

        @pl.when(i > 0)
        def _():
          drain(bufb, semsb)

        gather(t0 + 1, bufb, semgb)
        drain(bufa, semga)
        scatter(t0, bufa, semsa)

        @pl.when(t0 + 2 < kb)
        def _():
          drain(bufa, semsa)
          gather(t0 + 2, bufa, semga)

        drain(bufb, semgb)
        scatter(t0 + 1, bufb, semsb)
        return 0

      lax.fori_loop(0, kb // 2, pair, 0)
      drain(bufa, semsa)
      drain(bufb, semsb)
      return 0

    @pl.when(c == 0)
    def _():
      lax.fori_loop(0, k0 // kb, batch, 0)

    @pl.when(c == 1)
    def _():
      lax.fori_loop(0, (kt - k0) // kb, batch, 0)

    plsc.subcore_barrier()

    for z in range(nz):
      pltpu.sync_copy(acc.at[pl.ds(base + z * CH, CH)], bufa)
      pltpu.sync_copy(bufa, psum.at[c, pl.ds(base + z * CH, CH)])

  return sck


@functools.lru_cache(maxsize=None)
def _sc_counts(nr, kt, kb, k0):
  rpt = nr // NS
  nz = rpt // CH
  mesh = plsc.VectorSubcoreMesh(
      core_axis_name="c", subcore_axis_name="s", num_cores=NC,
      num_subcores=NS)

  @functools.partial(
      pl.kernel,
      out_type=(
          jax.ShapeDtypeStruct((NC, nr, 16), jnp.float32),
          jax.ShapeDtypeStruct((NC, nr, 16), jnp.float32),
      ),
      mesh=mesh,
      compiler_params=pltpu.CompilerParams(use_tc_tiling_on_sc=False),
      scratch_types=[
          pltpu.VMEM((kb, CH), jnp.int32),
          pltpu.VMEM((kb, CH), jnp.int32),
          pltpu.VMEM((CH, 16), jnp.float32),
          pltpu.VMEM((CH, 16), jnp.float32),
          pltpu.VMEM_SHARED((nr, 16), jnp.float32),
          pltpu.VMEM_SHARED((nr, 16), jnp.float32),
      ],
  )
  def sck(didx, sidx, cntd, cnts, dv, sv, ones, zrow, accd, accs):
    c = lax.axis_index("c")
    s = lax.axis_index("s")
    koff = c * k0

    def fill_row(i, _):
      ones[i] = jnp.full((16,), 1.0, jnp.float32)
      zrow[i] = jnp.zeros((16,), jnp.float32)
      return 0

    lax.fori_loop(0, CH, fill_row, 0)

    base = s * rpt
    for z in range(nz):
      pltpu.sync_copy(zrow, accd.at[pl.ds(base + z * CH, CH)])
      pltpu.sync_copy(zrow, accs.at[pl.ds(base + z * CH, CH)])
    plsc.subcore_barrier()

    def step(j, _):
      pltpu.sync_copy(ones, accd.at[dv.at[j]], add=True)
      pltpu.sync_copy(ones, accs.at[sv.at[j]], add=True)
      return 0

    def batch(b, _):
      pltpu.sync_copy(didx.at[s, pl.ds(koff + b * kb, kb)], dv)
      pltpu.sync_copy(sidx.at[s, pl.ds(koff + b * kb, kb)], sv)
      lax.fori_loop(0, kb, step, 0)
      return 0

    @pl.when(c == 0)
    def _():
      lax.fori_loop(0, k0 // kb, batch, 0)

    @pl.when(c == 1)
    def _():
      lax.fori_loop(0, (kt - k0) // kb, batch, 0)

    plsc.subcore_barrier()

    for z in range(nz):
      pltpu.sync_copy(accd.at[pl.ds(base + z * CH, CH)], ones)
      pltpu.sync_copy(ones, cntd.at[c, pl.ds(base + z * CH, CH)])
      pltpu.sync_copy(accs.at[pl.ds(base + z * CH, CH)], zrow)
      pltpu.sync_copy(zrow, cnts.at[c, pl.ds(base + z * CH, CH)])

  return sck


def _combine_norm_call(p, pc):
  nr, d = p.shape[1], p.shape[2]
  br = 1024

  def body(p_ref, c_ref, o_ref):
    ssum = p_ref[0] + p_ref[1]
    cnt = c_ref[0, :, 0:1] + c_ref[1, :, 0:1]
    o_ref[...] = ssum / (cnt + EPS)

  return pl.pallas_call(
      body,
      grid=(nr // br,),
      in_specs=[
          pl.BlockSpec((NC, br, d), lambda i: (0, i, 0)),
          pl.BlockSpec((NC, br, 16), lambda i: (0, i, 0)),
      ],
      out_specs=pl.BlockSpec((br, d), lambda i: (i, 0)),
      out_shape=jax.ShapeDtypeStruct((nr, d), jnp.float32),
  )(p, pc)


def _final_call(q, qc, xn, bias2d):
  nr, d = q.shape[1], q.shape[2]
  br = 1024

  def body(q_ref, c_ref, x_ref, b_ref, o_ref):
    ssum = q_ref[0] + q_ref[1]
    deg = c_ref[0, :, 0:1] + c_ref[1, :, 0:1]
    o_ref[...] = ssum / (deg + EPS) + x_ref[...] + b_ref[...]

  return pl.pallas_call(
      body,
      grid=(nr // br,),
      in_specs=[
          pl.BlockSpec((NC, br, d), lambda i: (0, i, 0)),
          pl.BlockSpec((NC, br, 16), lambda i: (0, i, 0)),
          pl.BlockSpec((br, d), lambda i: (i, 0)),
          pl.BlockSpec((1, d), lambda i: (0, 0)),
      ],
      out_specs=pl.BlockSpec((br, d), lambda i: (i, 0)),
      out_shape=jax.ShapeDtypeStruct((nr, d), jnp.float32),
  )(q, qc, xn, bias2d)


def kernel(x, hyperedge_index, W_node, W_edge, bias):
  n, d = x.shape
  src = hyperedge_index[0].astype(jnp.int32)
  dst = hyperedge_index[1].astype(jnp.int32)
  ni = src.shape[0]

  nr = -(-(n + 1) // (NS * CH)) * (NS * CH)
  kb = 32
  kt = -(-ni // (NS * CH * kb)) * kb
  k0 = (kt * 4 // 5) // kb * kb
  npad = NS * kt * CH - ni
  src_p = jnp.concatenate(
      [src, jnp.full((npad,), n, jnp.int32)]).reshape(NS, kt, CH)
  dst_p = jnp.concatenate(
      [dst, jnp.full((npad,), n, jnp.int32)]).reshape(NS, kt, CH)

  xp = jnp.zeros((nr, d), jnp.float32).at[:n].set(x)
  xn = _matmul_call(xp, W_node)

  sck = _sc_gather_scatter(nr, d, kt, kb, k0)
  p = sck(xn, src_p, dst_p)
  cnt, deg = _sc_counts(nr, kt, kb, kt // 2)(dst_p, src_p)
  attr2 = _combine_norm_call(p, cnt)
  q = sck(attr2, dst_p, src_p)
  out = _final_call(q, deg, xn, bias.reshape(1, d))
  return out[:n]
